# baseline (speedup 1.0000x reference)
import functools

import jax
import jax.numpy as jnp
from jax import lax
from jax.experimental import pallas as pl
from jax.experimental.pallas import tpu as pltpu
from jax.experimental.pallas import tpu_sc as plsc

N = 10000
E = 160000
TAU0 = 0.1

NS = 16
NP = 10112
ROWS_PT = NP // NS
EP = 163840
K = 128
K1 = 40
EPT1 = EP // NS
NCH1 = EPT1 // K1
EPT2 = EP // (2 * NS)
NCH2 = EPT2 // K
EROWS = EP // K
EROWS1 = EP // K1
NCH5 = EPT2 // K1


def _ring4(nch, issue_gather, wait_gather, scatter_start, scatter_wait):
    for i in range(min(4, nch)):
        issue_gather(i, i)

    def quad(i4, carry):
        for q in range(4):
            i = i4 * 4 + q
            nb = (q + 1) % 4

            @pl.when(i >= 3)
            def _prep(i=i, nb=nb):
                scatter_wait(i - 3, nb)

                @pl.when(i + 1 < nch)
                def _ig(i=i, nb=nb):
                    issue_gather(i + 1, nb)

            wait_gather(i, q)
            scatter_start(i, q)
        return carry

    lax.fori_loop(0, nch // 4, quad, 0)
    for k in range(3):
        i = nch - 3 + k
        scatter_wait(i, i % 4)

_mesh = plsc.VectorSubcoreMesh(core_axis_name="c", subcore_axis_name="s")
_sc_params = pltpu.CompilerParams(use_tc_tiling_on_sc=False,
                                  needs_layout_passes=False)


def _p1_body(t1a, t1b, sidx, didx, zrows, zcnt, onesr, agg_o, cnt_o,
             src_all, dst_all, b0, b1, b2, b3, ones_v, acc, accc,
             g0, g1, g2, g3, s0, s1, s2, s3):
    c = lax.axis_index("c")
    s = lax.axis_index("s")
    r0 = s * ROWS_PT
    bufs = [b0, b1, b2, b3]
    gsem = [g0, g1, g2, g3]
    ssem = [s0, s1, s2, s3]
    pltpu.sync_copy(zrows, acc.at[pl.ds(r0, ROWS_PT)])
    pltpu.sync_copy(zcnt, accc.at[pl.ds(r0, ROWS_PT)])
    pltpu.sync_copy(onesr, ones_v)
    pltpu.sync_copy(sidx.at[pl.ds(s * NCH1, NCH1)], src_all)
    pltpu.sync_copy(didx.at[pl.ds(s * NCH1, NCH1)], dst_all)
    plsc.subcore_barrier()

    def issue_gather(i, q):
        @pl.when(c == 0)
        def _a():
            pltpu.async_copy(t1a.at[src_all.at[i]], bufs[q], gsem[q])

        @pl.when(c == 1)
        def _b():
            pltpu.async_copy(t1b.at[src_all.at[i]], bufs[q], gsem[q])

    def wait_gather(i, q):
        pltpu.make_async_copy(t1a.at[src_all.at[i]], bufs[q], gsem[q]).wait()

    def scatter_start(i, q):
        pltpu.async_copy(bufs[q], acc.at[dst_all.at[i]], ssem[q], add=True)
        pltpu.async_copy(ones_v, accc.at[dst_all.at[i]], ssem[q], add=True)

    def scatter_wait(i, q):
        pltpu.make_async_copy(bufs[q], acc.at[dst_all.at[i]], ssem[q]).wait()
        pltpu.make_async_copy(ones_v, accc.at[dst_all.at[i]], ssem[q]).wait()

    _ring4(NCH1, issue_gather, wait_gather, scatter_start, scatter_wait)
    plsc.subcore_barrier()
    pltpu.sync_copy(acc.at[pl.ds(r0, ROWS_PT)], agg_o.at[c, pl.ds(r0, ROWS_PT)])
    pltpu.sync_copy(accc.at[pl.ds(r0, ROWS_PT)], cnt_o.at[c, pl.ds(r0, ROWS_PT)])


_sc_pass1 = pl.kernel(
    _p1_body, mesh=_mesh, compiler_params=_sc_params,
    out_type=[jax.ShapeDtypeStruct((2, NP, 128), jnp.float32),
              jax.ShapeDtypeStruct((2, NP, 8), jnp.float32)],
    scratch_types=(
        [pltpu.VMEM((NCH1, K1), jnp.int32),
         pltpu.VMEM((NCH1, K1), jnp.int32)]
        + [pltpu.VMEM((K1, 128), jnp.float32)] * 4
        + [pltpu.VMEM((K1, 8), jnp.float32),
           pltpu.VMEM_SHARED((NP, 128), jnp.float32),
           pltpu.VMEM_SHARED((NP, 8), jnp.float32)]
        + [pltpu.SemaphoreType.DMA] * 8
    ))


def _p3_body(qtab, sidx, didx, zq, sq_o, src_all, dst_all, b0, b1, b2, b3,
             acc, g0, g1, g2, g3, s0, s1, s2, s3):
    c = lax.axis_index("c")
    s = lax.axis_index("s")
    r0 = s * ROWS_PT
    w = c * NS + s
    bufs = [b0, b1, b2, b3]
    gsem = [g0, g1, g2, g3]
    ssem = [s0, s1, s2, s3]
    pltpu.sync_copy(zq, acc.at[pl.ds(r0, ROWS_PT)])
    pltpu.sync_copy(sidx.at[pl.ds(w * NCH2, NCH2)], src_all)
    pltpu.sync_copy(didx.at[pl.ds(w * NCH2, NCH2)], dst_all)
    plsc.subcore_barrier()

    def issue_gather(i, q):
        pltpu.async_copy(qtab.at[src_all.at[i]], bufs[q], gsem[q])

    def wait_gather(i, q):
        pltpu.make_async_copy(qtab.at[src_all.at[i]], bufs[q], gsem[q]).wait()

    def scatter_start(i, q):
        pltpu.async_copy(bufs[q], acc.at[dst_all.at[i]], ssem[q], add=True)

    def scatter_wait(i, q):
        pltpu.make_async_copy(bufs[q], acc.at[dst_all.at[i]], ssem[q]).wait()

    _ring4(NCH2, issue_gather, wait_gather, scatter_start, scatter_wait)
    plsc.subcore_barrier()
    pltpu.sync_copy(acc.at[pl.ds(r0, ROWS_PT)], sq_o.at[c, pl.ds(r0, ROWS_PT)])


_sc_pass3 = pl.kernel(
    _p3_body, mesh=_mesh, compiler_params=_sc_params,
    out_type=[jax.ShapeDtypeStruct((2, NP, 16), jnp.float32)],
    scratch_types=(
        [pltpu.VMEM((NCH2, K), jnp.int32),
         pltpu.VMEM((NCH2, K), jnp.int32)]
        + [pltpu.VMEM((K, 16), jnp.float32)] * 4
        + [pltpu.VMEM_SHARED((NP, 16), jnp.float32)]
        + [pltpu.SemaphoreType.DMA] * 8
    ))


def _p5a_body(ytab, sidx, didx, zrows, ya_o,
              src_all, dst_all, b0, b1, b2, b3, acc,
              g0, g1, g2, g3, s0, s1, s2, s3):
    c = lax.axis_index("c")
    s = lax.axis_index("s")
    r0 = s * ROWS_PT
    w = c * NS + s
    bufs = [b0, b1, b2, b3]
    gsem = [g0, g1, g2, g3]
    ssem = [s0, s1, s2, s3]
    pltpu.sync_copy(zrows, acc.at[pl.ds(r0, ROWS_PT)])
    pltpu.sync_copy(sidx.at[pl.ds(w * NCH5, NCH5)], src_all)
    pltpu.sync_copy(didx.at[pl.ds(w * NCH5, NCH5)], dst_all)
    plsc.subcore_barrier()

    def issue_gather(i, q):
        pltpu.async_copy(ytab.at[src_all.at[i]], bufs[q], gsem[q])

    def wait_gather(i, q):
        pltpu.make_async_copy(ytab.at[src_all.at[i]], bufs[q], gsem[q]).wait()

    def scatter_start(i, q):
        pltpu.async_copy(bufs[q], acc.at[dst_all.at[i]], ssem[q], add=True)

    def scatter_wait(i, q):
        pltpu.make_async_copy(bufs[q], acc.at[dst_all.at[i]], ssem[q]).wait()

    _ring4(NCH5, issue_gather, wait_gather, scatter_start, scatter_wait)
    plsc.subcore_barrier()
    pltpu.sync_copy(acc.at[pl.ds(r0, ROWS_PT)], ya_o.at[c, pl.ds(r0, ROWS_PT)])


_sc_pass5a = pl.kernel(
    _p5a_body, mesh=_mesh, compiler_params=_sc_params,
    out_type=[jax.ShapeDtypeStruct((2, NP, 256), jnp.bfloat16)],
    scratch_types=(
        [pltpu.VMEM((NCH5, K1), jnp.int32),
         pltpu.VMEM((NCH5, K1), jnp.int32)]
        + [pltpu.VMEM((K1, 256), jnp.bfloat16)] * 4
        + [pltpu.VMEM_SHARED((NP, 256), jnp.bfloat16)]
        + [pltpu.SemaphoreType.DMA] * 8
    ))


def _p5b_body(sidx, didx, ptab, eat, zea, eaa_o,
              esrc_all, edst_all, ea0, ea1, st0, st1, ptab_v, accea,
              e0s, e1s, s0s, s1s):
    c = lax.axis_index("c")
    s = lax.axis_index("s")
    r0 = s * ROWS_PT
    w = c * NS + s
    eat_v = [ea0, ea1]
    stage = [st0, st1]
    esem = [e0s, e1s]
    ssem = [s0s, s1s]
    pltpu.sync_copy(zea, accea.at[pl.ds(r0, ROWS_PT)])
    pltpu.sync_copy(zea.at[pl.ds(0, K)], st0)
    pltpu.sync_copy(zea.at[pl.ds(0, K)], st1)
    pltpu.sync_copy(ptab, ptab_v)
    pltpu.sync_copy(sidx.at[pl.ds(w * NCH2, NCH2)], esrc_all)
    pltpu.sync_copy(didx.at[pl.ds(w * NCH2, NCH2)], edst_all)
    plsc.subcore_barrier()

    iot = lax.iota(jnp.int32, 16)
    base_e = w * EPT2

    def eload(g, q):
        pltpu.async_copy(eat.at[:, pl.ds(base_e + g * K, K)], eat_v[q],
                         esem[q])

    def ewait(g, q):
        pltpu.make_async_copy(eat.at[:, pl.ds(base_e + g * K, K)], eat_v[q],
                              esem[q]).wait()

    def swait(i, q):
        pltpu.make_async_copy(stage[q], accea.at[edst_all.at[i]],
                              ssem[q]).wait()

    def compute(g, q):
        def group(g2, carry2):
            b = g2 * 16
            idxv = esrc_all[g, pl.ds(b, 16)]
            pv = plsc.load_gather(ptab_v, [idxv])
            rows16 = b + iot
            for j in range(16):
                colv = eat_v[q][j, pl.ds(b, 16)]
                plsc.store_scatter(
                    stage[q], [rows16, jnp.full((16,), j, jnp.int32)],
                    colv * pv)
            plsc.store_scatter(
                stage[q], [rows16, jnp.full((16,), 16, jnp.int32)], pv)
            return carry2

        lax.fori_loop(0, K // 16, group, 0)
        pltpu.async_copy(stage[q], accea.at[edst_all.at[g]], ssem[q],
                         add=True)

    eload(0, 0)

    def pair(p, carry):
        g0 = 2 * p
        eload(g0 + 1, 1)
        ewait(g0, 0)

        @pl.when(g0 >= 2)
        def _w0():
            swait(g0 - 2, 0)

        compute(g0, 0)

        @pl.when(g0 + 2 < NCH2)
        def _l2():
            eload(g0 + 2, 0)

        ewait(g0 + 1, 1)

        @pl.when(g0 >= 1)
        def _w1():
            swait(g0 - 1, 1)

        compute(g0 + 1, 1)
        return carry

    lax.fori_loop(0, NCH2 // 2, pair, 0)
    swait(NCH2 - 2, 0)
    swait(NCH2 - 1, 1)
    plsc.subcore_barrier()
    pltpu.sync_copy(accea.at[pl.ds(r0, ROWS_PT)], eaa_o.at[c, pl.ds(r0, ROWS_PT)])


_sc_pass5b = pl.kernel(
    _p5b_body, mesh=_mesh, compiler_params=_sc_params,
    out_type=[jax.ShapeDtypeStruct((2, NP, 32), jnp.float32)],
    scratch_types=(
        [pltpu.VMEM((NCH2, K), jnp.int32),
         pltpu.VMEM((NCH2, K), jnp.int32)]
        + [pltpu.VMEM((16, K), jnp.float32)] * 2
        + [pltpu.VMEM((K, 32), jnp.float32)] * 2
        + [pltpu.VMEM((N,), jnp.float32),
           pltpu.VMEM_SHARED((NP, 32), jnp.float32)]
        + [pltpu.SemaphoreType.DMA] * 4
    ))


BN = 1000
GRID = N // BN


def _dot(a, b):
    return lax.dot_general(a, b, (((1,), (0,)), ((), ())),
                           precision=lax.Precision.HIGHEST,
                           preferred_element_type=jnp.float32)


def _tc2_body(x_r, agg_r, cnt_r, w1_r, w2_r, w3_r, w4_r, wt_r, we_r,
              q_r, v_r):
    x = x_r[...]
    agg = agg_r[...]
    A = (jnp.concatenate([agg[0], agg[1]], axis=1) + x) \
        / (cnt_r[...][0][:, 0:1] + 1.0)
    hin = jnp.maximum(_dot(A, w1_r[...]), 0.0)
    hout = jnp.maximum(_dot(A, w3_r[...]), 0.0)
    qin = _dot(hin, w2_r[...])
    qout = _dot(hout, w4_r[...])
    tz = _dot(A, wt_r[...])
    temp = jnp.maximum(tz, 0.0) + jnp.log(1.0 + jnp.exp(-jnp.abs(tz))) + TAU0
    q_r[...] = jnp.concatenate(
        [qin, qout, temp, jnp.zeros((BN, 11), jnp.float32)], axis=1)
    v_r[...] = _dot(x, we_r[...])


def _tc4_body(q_r, sq_r, cnt_r, gd_r, v_r, y_r, p_r):
    Qb = q_r[...]
    sq = sq_r[...]
    SQ = sq[0] + sq[1]
    d1 = cnt_r[...][0][:, 0:1] + 1.0
    gd = gd_r[...]
    temp = Qb[:, 4:5]
    zin = ((SQ[:, 0:1] + Qb[:, 0:1] - SQ[:, 1:2] - Qb[:, 1:2]) / d1
           + gd[:, 0:1]) / temp
    zout = ((SQ[:, 2:3] + Qb[:, 2:3] - SQ[:, 3:4] - Qb[:, 3:4]) / d1
            + gd[:, 1:2]) / temp
    pin = 1.0 / (1.0 + jnp.exp(-zin))
    pout = 1.0 / (1.0 + jnp.exp(-zout))
    y_r[...] = (pout * v_r[...]).astype(jnp.bfloat16)
    p_r[...] = jnp.concatenate(
        [pout, pin, jnp.zeros((BN, 6), jnp.float32)], axis=1)


def _tc6_body(ya_r, ea_r, p_r, v_r, wedge_r, wenv_r, be_r, lg_r, lb_r, o_r):
    ya = ya_r[...].astype(jnp.float32)
    Sv = ya[0] + ya[1]
    ea = ea_r[...]
    ea2 = ea[0] + ea[1]
    Sea = ea2[:, 0:16]
    Sp = ea2[:, 16:17]
    pin = p_r[...][:, 1:2]
    wee = _dot(wedge_r[...], wenv_r[...])
    num = pin * (Sv + _dot(Sea, wee)) + v_r[...]
    h = num / (pin * Sp + 1.0) + be_r[...]
    mu = jnp.mean(h, axis=1, keepdims=True)
    var = jnp.mean((h - mu) ** 2, axis=1, keepdims=True)
    o_r[...] = (h - mu) / jnp.sqrt(var + 1e-5) * lg_r[...] + lb_r[...]


def _full(shape):
    return pl.BlockSpec(shape, lambda i: tuple(0 for _ in shape))


def _rows(shape):
    if len(shape) == 2:
        return pl.BlockSpec(shape, lambda i: (i, 0))
    return pl.BlockSpec(shape, lambda i: (0, i, 0))


_tc_pass2 = pl.pallas_call(
    _tc2_body,
    grid=(GRID,),
    in_specs=[_rows((BN, 256)), _rows((2, BN, 128)), _rows((1, BN, 8)),
              _full((256, 256)), _full((256, 2)), _full((256, 256)),
              _full((256, 2)), _full((256, 1)), _full((256, 256))],
    out_specs=[_rows((BN, 16)), _rows((BN, 256))],
    out_shape=[jax.ShapeDtypeStruct((N, 16), jnp.float32),
               jax.ShapeDtypeStruct((N, 256), jnp.float32)],
)

_tc_pass4 = pl.pallas_call(
    _tc4_body,
    grid=(GRID,),
    in_specs=[_rows((BN, 16)), _rows((2, BN, 16)), _rows((1, BN, 8)),
              _rows((BN, 8)), _rows((BN, 256))],
    out_specs=[_rows((BN, 256)), _rows((BN, 8))],
    out_shape=[jax.ShapeDtypeStruct((N, 256), jnp.bfloat16),
               jax.ShapeDtypeStruct((N, 8), jnp.float32)],
)

_tc_pass6 = pl.pallas_call(
    _tc6_body,
    grid=(GRID,),
    in_specs=[_rows((2, BN, 256)), _rows((2, BN, 32)), _rows((BN, 8)),
              _rows((BN, 256)), _full((16, 256)), _full((256, 256)),
              _full((1, 256)), _full((1, 256)), _full((1, 256))],
    out_specs=[_rows((BN, 256))],
    out_shape=[jax.ShapeDtypeStruct((N, 256), jnp.float32)],
)


def kernel(x, edge_index, edge_attr, W_env, b_env, W_edge, W_in1, W_in2,
           W_out1, W_out2, W_temp, ln_gamma, ln_beta):
    f32 = jnp.float32
    src = edge_index[0]
    dst = edge_index[1]
    pad = EP - E
    srcp = jnp.concatenate([src, jnp.zeros((pad,), jnp.int32)])
    dstp = jnp.concatenate([dst, jnp.full((pad,), N, jnp.int32)])
    sidx = srcp.reshape(EROWS, K)
    didx = dstp.reshape(EROWS, K)
    sidx1 = srcp.reshape(EROWS1, K1)
    didx1 = dstp.reshape(EROWS1, K1)

    zrows = jnp.zeros((ROWS_PT, 128), f32)
    zcnt = jnp.zeros((ROWS_PT, 8), f32)
    zq = jnp.zeros((ROWS_PT, 16), f32)
    zea = jnp.zeros((ROWS_PT, 32), f32)
    onesr = jnp.concatenate(
        [jnp.ones((K1, 1), f32), jnp.zeros((K1, 7), f32)], axis=1)

    agg_s, cnt_s = _sc_pass1(x[:, :128], x[:, 128:], sidx1, didx1,
                             zrows, zcnt, onesr)

    Q, v = _tc_pass2(x, agg_s, cnt_s, W_in1, W_in2, W_out1,
                     W_out2, W_temp, W_env)

    (sq_s,) = _sc_pass3(Q, sidx, didx, zq)

    gkey = jax.random.key(42)
    k1, k2 = jax.random.split(gkey)
    u_in = jax.random.uniform(k1, (N, 2), f32, 1e-6, 1.0 - 1e-6)
    u_out = jax.random.uniform(k2, (N, 2), f32, 1e-6, 1.0 - 1e-6)
    g_in = -jnp.log(-jnp.log(u_in))
    g_out = -jnp.log(-jnp.log(u_out))
    gd = jnp.concatenate(
        [(g_in[:, 0] - g_in[:, 1])[:, None],
         (g_out[:, 0] - g_out[:, 1])[:, None],
         jnp.zeros((N, 6), f32)], axis=1)
    ytab, P = _tc_pass4(Q, sq_s, cnt_s, gd, v)

    ptab = P[:, 0]
    eat = jnp.pad(edge_attr, ((0, pad), (0, 0))).T
    (ya_s,) = _sc_pass5a(ytab, sidx1, didx1,
                         jnp.zeros((ROWS_PT, 256), jnp.bfloat16))
    (eaa_s,) = _sc_pass5b(sidx, didx, ptab, eat, zea)

    out = _tc_pass6(ya_s, eaa_s, P, v, W_edge, W_env,
                    b_env.reshape(1, 256), ln_gamma.reshape(1, 256),
                    ln_beta.reshape(1, 256))
    return out[0]


# speedup vs baseline: 1.0713x; 1.0713x over previous
import functools

import jax
import jax.numpy as jnp
from jax import lax
from jax.experimental import pallas as pl
from jax.experimental.pallas import tpu as pltpu
from jax.experimental.pallas import tpu_sc as plsc

N = 10000
E = 160000
TAU0 = 0.1

NS = 16
NP = 10112
ROWS_PT = NP // NS
EP = 163840
K = 128
K1 = 40
EPT1 = EP // NS
NCH1 = EPT1 // K1
EPT2 = EP // (2 * NS)
NCH2 = EPT2 // K
EROWS = EP // K
EROWS1 = EP // K1
NCH5 = EPT2 // K1


def _ring4(nch, issue_gather, wait_gather, scatter_start, scatter_wait):
    for i in range(min(4, nch)):
        issue_gather(i, i)

    def quad(i4, carry):
        for q in range(4):
            i = i4 * 4 + q
            nb = (q + 1) % 4

            @pl.when(i >= 3)
            def _prep(i=i, nb=nb):
                scatter_wait(i - 3, nb)

                @pl.when(i + 1 < nch)
                def _ig(i=i, nb=nb):
                    issue_gather(i + 1, nb)

            wait_gather(i, q)
            scatter_start(i, q)
        return carry

    lax.fori_loop(0, nch // 4, quad, 0)
    for k in range(3):
        i = nch - 3 + k
        scatter_wait(i, i % 4)

_mesh = plsc.VectorSubcoreMesh(core_axis_name="c", subcore_axis_name="s")
_sc_params = pltpu.CompilerParams(use_tc_tiling_on_sc=False,
                                  needs_layout_passes=False)


def _p1_body(t1a, t1b, sidx, didx, zrows, zcnt, onesr, agg_o, cnt_o,
             src_all, dst_all, b0, b1, b2, b3, ones_v, acc, accc,
             g0, g1, g2, g3, s0, s1, s2, s3):
    c = lax.axis_index("c")
    s = lax.axis_index("s")
    r0 = s * ROWS_PT
    bufs = [b0, b1, b2, b3]
    gsem = [g0, g1, g2, g3]
    ssem = [s0, s1, s2, s3]
    pltpu.sync_copy(zrows, acc.at[pl.ds(r0, ROWS_PT)])
    pltpu.sync_copy(zcnt, accc.at[pl.ds(r0, ROWS_PT)])
    pltpu.sync_copy(onesr, ones_v)
    pltpu.sync_copy(sidx.at[pl.ds(s * NCH1, NCH1)], src_all)
    pltpu.sync_copy(didx.at[pl.ds(s * NCH1, NCH1)], dst_all)
    plsc.subcore_barrier()

    def issue_gather(i, q):
        @pl.when(c == 0)
        def _a():
            pltpu.async_copy(t1a.at[src_all.at[i]], bufs[q], gsem[q])

        @pl.when(c == 1)
        def _b():
            pltpu.async_copy(t1b.at[src_all.at[i]], bufs[q], gsem[q])

    def wait_gather(i, q):
        pltpu.make_async_copy(t1a.at[src_all.at[i]], bufs[q], gsem[q]).wait()

    def scatter_start(i, q):
        pltpu.async_copy(bufs[q], acc.at[dst_all.at[i]], ssem[q], add=True)
        pltpu.async_copy(ones_v, accc.at[dst_all.at[i]], ssem[q], add=True)

    def scatter_wait(i, q):
        pltpu.make_async_copy(bufs[q], acc.at[dst_all.at[i]], ssem[q]).wait()
        pltpu.make_async_copy(ones_v, accc.at[dst_all.at[i]], ssem[q]).wait()

    _ring4(NCH1, issue_gather, wait_gather, scatter_start, scatter_wait)
    plsc.subcore_barrier()
    pltpu.sync_copy(acc.at[pl.ds(r0, ROWS_PT)], agg_o.at[c, pl.ds(r0, ROWS_PT)])
    pltpu.sync_copy(accc.at[pl.ds(r0, ROWS_PT)], cnt_o.at[c, pl.ds(r0, ROWS_PT)])


_sc_pass1 = pl.kernel(
    _p1_body, mesh=_mesh, compiler_params=_sc_params,
    out_type=[jax.ShapeDtypeStruct((2, NP, 128), jnp.float32),
              jax.ShapeDtypeStruct((2, NP, 8), jnp.float32)],
    scratch_types=(
        [pltpu.VMEM((NCH1, K1), jnp.int32),
         pltpu.VMEM((NCH1, K1), jnp.int32)]
        + [pltpu.VMEM((K1, 128), jnp.float32)] * 4
        + [pltpu.VMEM((K1, 8), jnp.float32),
           pltpu.VMEM_SHARED((NP, 128), jnp.float32),
           pltpu.VMEM_SHARED((NP, 8), jnp.float32)]
        + [pltpu.SemaphoreType.DMA] * 8
    ))


def _p3_body(qtab, sidx, didx, zq, sq_o, src_all, dst_all, b0, b1, b2, b3,
             acc, g0, g1, g2, g3, s0, s1, s2, s3):
    c = lax.axis_index("c")
    s = lax.axis_index("s")
    r0 = s * ROWS_PT
    w = c * NS + s
    bufs = [b0, b1, b2, b3]
    gsem = [g0, g1, g2, g3]
    ssem = [s0, s1, s2, s3]
    pltpu.sync_copy(zq, acc.at[pl.ds(r0, ROWS_PT)])
    pltpu.sync_copy(sidx.at[pl.ds(w * NCH2, NCH2)], src_all)
    pltpu.sync_copy(didx.at[pl.ds(w * NCH2, NCH2)], dst_all)
    plsc.subcore_barrier()

    def issue_gather(i, q):
        pltpu.async_copy(qtab.at[src_all.at[i]], bufs[q], gsem[q])

    def wait_gather(i, q):
        pltpu.make_async_copy(qtab.at[src_all.at[i]], bufs[q], gsem[q]).wait()

    def scatter_start(i, q):
        pltpu.async_copy(bufs[q], acc.at[dst_all.at[i]], ssem[q], add=True)

    def scatter_wait(i, q):
        pltpu.make_async_copy(bufs[q], acc.at[dst_all.at[i]], ssem[q]).wait()

    _ring4(NCH2, issue_gather, wait_gather, scatter_start, scatter_wait)
    plsc.subcore_barrier()
    pltpu.sync_copy(acc.at[pl.ds(r0, ROWS_PT)], sq_o.at[c, pl.ds(r0, ROWS_PT)])


_sc_pass3 = pl.kernel(
    _p3_body, mesh=_mesh, compiler_params=_sc_params,
    out_type=[jax.ShapeDtypeStruct((2, NP, 16), jnp.float32)],
    scratch_types=(
        [pltpu.VMEM((NCH2, K), jnp.int32),
         pltpu.VMEM((NCH2, K), jnp.int32)]
        + [pltpu.VMEM((K, 16), jnp.float32)] * 4
        + [pltpu.VMEM_SHARED((NP, 16), jnp.float32)]
        + [pltpu.SemaphoreType.DMA] * 8
    ))


def _p5a_body(yta, ytb, sidx, didx, zrows, ya_o,
              src_all, dst_all, b0, b1, b2, b3, acc,
              g0, g1, g2, g3, s0, s1, s2, s3):
    c = lax.axis_index("c")
    s = lax.axis_index("s")
    r0 = s * ROWS_PT
    bufs = [b0, b1, b2, b3]
    gsem = [g0, g1, g2, g3]
    ssem = [s0, s1, s2, s3]
    pltpu.sync_copy(zrows, acc.at[pl.ds(r0, ROWS_PT)])
    pltpu.sync_copy(sidx.at[pl.ds(s * NCH1, NCH1)], src_all)
    pltpu.sync_copy(didx.at[pl.ds(s * NCH1, NCH1)], dst_all)
    plsc.subcore_barrier()

    def issue_gather(i, q):
        @pl.when(c == 0)
        def _a():
            pltpu.async_copy(yta.at[src_all.at[i]], bufs[q], gsem[q])

        @pl.when(c == 1)
        def _b():
            pltpu.async_copy(ytb.at[src_all.at[i]], bufs[q], gsem[q])

    def wait_gather(i, q):
        pltpu.make_async_copy(yta.at[src_all.at[i]], bufs[q], gsem[q]).wait()

    def scatter_start(i, q):
        pltpu.async_copy(bufs[q], acc.at[dst_all.at[i]], ssem[q], add=True)

    def scatter_wait(i, q):
        pltpu.make_async_copy(bufs[q], acc.at[dst_all.at[i]], ssem[q]).wait()

    _ring4(NCH1, issue_gather, wait_gather, scatter_start, scatter_wait)
    plsc.subcore_barrier()
    pltpu.sync_copy(acc.at[pl.ds(r0, ROWS_PT)], ya_o.at[c, pl.ds(r0, ROWS_PT)])


_sc_pass5a = pl.kernel(
    _p5a_body, mesh=_mesh, compiler_params=_sc_params,
    out_type=[jax.ShapeDtypeStruct((2, NP, 128), jnp.bfloat16)],
    scratch_types=(
        [pltpu.VMEM((NCH1, K1), jnp.int32),
         pltpu.VMEM((NCH1, K1), jnp.int32)]
        + [pltpu.VMEM((K1, 128), jnp.bfloat16)] * 4
        + [pltpu.VMEM_SHARED((NP, 128), jnp.bfloat16)]
        + [pltpu.SemaphoreType.DMA] * 8
    ))


def _p5b_body(sidx, didx, ptab, eat, zea, eaa_o,
              esrc_all, edst_all, ea0, ea1, st0, st1, ptab_v, accea,
              e0s, e1s, s0s, s1s):
    c = lax.axis_index("c")
    s = lax.axis_index("s")
    r0 = s * ROWS_PT
    w = c * NS + s
    eat_v = [ea0, ea1]
    stage = [st0, st1]
    esem = [e0s, e1s]
    ssem = [s0s, s1s]
    pltpu.sync_copy(zea, accea.at[pl.ds(r0, ROWS_PT)])
    pltpu.sync_copy(zea.at[pl.ds(0, K)], st0)
    pltpu.sync_copy(zea.at[pl.ds(0, K)], st1)
    pltpu.sync_copy(ptab, ptab_v)
    pltpu.sync_copy(sidx.at[pl.ds(w * NCH2, NCH2)], esrc_all)
    pltpu.sync_copy(didx.at[pl.ds(w * NCH2, NCH2)], edst_all)
    plsc.subcore_barrier()

    iot = lax.iota(jnp.int32, 16)
    base_e = w * EPT2

    def eload(g, q):
        pltpu.async_copy(eat.at[:, pl.ds(base_e + g * K, K)], eat_v[q],
                         esem[q])

    def ewait(g, q):
        pltpu.make_async_copy(eat.at[:, pl.ds(base_e + g * K, K)], eat_v[q],
                              esem[q]).wait()

    def swait(i, q):
        pltpu.make_async_copy(stage[q], accea.at[edst_all.at[i]],
                              ssem[q]).wait()

    def compute(g, q):
        def group(g2, carry2):
            b = g2 * 16
            idxv = esrc_all[g, pl.ds(b, 16)]
            pv = plsc.load_gather(ptab_v, [idxv])
            rows16 = b + iot
            for j in range(16):
                colv = eat_v[q][j, pl.ds(b, 16)]
                plsc.store_scatter(
                    stage[q], [rows16, jnp.full((16,), j, jnp.int32)],
                    colv * pv)
            plsc.store_scatter(
                stage[q], [rows16, jnp.full((16,), 16, jnp.int32)], pv)
            return carry2

        lax.fori_loop(0, K // 16, group, 0)
        pltpu.async_copy(stage[q], accea.at[edst_all.at[g]], ssem[q],
                         add=True)

    eload(0, 0)

    def pair(p, carry):
        g0 = 2 * p
        eload(g0 + 1, 1)
        ewait(g0, 0)

        @pl.when(g0 >= 2)
        def _w0():
            swait(g0 - 2, 0)

        compute(g0, 0)

        @pl.when(g0 + 2 < NCH2)
        def _l2():
            eload(g0 + 2, 0)

        ewait(g0 + 1, 1)

        @pl.when(g0 >= 1)
        def _w1():
            swait(g0 - 1, 1)

        compute(g0 + 1, 1)
        return carry

    lax.fori_loop(0, NCH2 // 2, pair, 0)
    swait(NCH2 - 2, 0)
    swait(NCH2 - 1, 1)
    plsc.subcore_barrier()
    pltpu.sync_copy(accea.at[pl.ds(r0, ROWS_PT)], eaa_o.at[c, pl.ds(r0, ROWS_PT)])


_sc_pass5b = pl.kernel(
    _p5b_body, mesh=_mesh, compiler_params=_sc_params,
    out_type=[jax.ShapeDtypeStruct((2, NP, 32), jnp.float32)],
    scratch_types=(
        [pltpu.VMEM((NCH2, K), jnp.int32),
         pltpu.VMEM((NCH2, K), jnp.int32)]
        + [pltpu.VMEM((16, K), jnp.float32)] * 2
        + [pltpu.VMEM((K, 32), jnp.float32)] * 2
        + [pltpu.VMEM((N,), jnp.float32),
           pltpu.VMEM_SHARED((NP, 32), jnp.float32)]
        + [pltpu.SemaphoreType.DMA] * 4
    ))


BN = 1000
GRID = N // BN


def _dot(a, b):
    return lax.dot_general(a, b, (((1,), (0,)), ((), ())),
                           precision=lax.Precision.HIGHEST,
                           preferred_element_type=jnp.float32)


def _tc2_body(x_r, agg_r, cnt_r, w1_r, w2_r, w3_r, w4_r, wt_r, we_r,
              q_r, v_r):
    x = x_r[...]
    agg = agg_r[...]
    A = (jnp.concatenate([agg[0], agg[1]], axis=1) + x) \
        / (cnt_r[...][0][:, 0:1] + 1.0)
    hin = jnp.maximum(_dot(A, w1_r[...]), 0.0)
    hout = jnp.maximum(_dot(A, w3_r[...]), 0.0)
    qin = _dot(hin, w2_r[...])
    qout = _dot(hout, w4_r[...])
    tz = _dot(A, wt_r[...])
    temp = jnp.maximum(tz, 0.0) + jnp.log(1.0 + jnp.exp(-jnp.abs(tz))) + TAU0
    q_r[...] = jnp.concatenate(
        [qin, qout, temp, jnp.zeros((BN, 11), jnp.float32)], axis=1)
    v_r[...] = _dot(x, we_r[...])


def _tc4_body(q_r, sq_r, cnt_r, gd_r, v_r, y_r, p_r):
    Qb = q_r[...]
    sq = sq_r[...]
    SQ = sq[0] + sq[1]
    d1 = cnt_r[...][0][:, 0:1] + 1.0
    gd = gd_r[...]
    temp = Qb[:, 4:5]
    zin = ((SQ[:, 0:1] + Qb[:, 0:1] - SQ[:, 1:2] - Qb[:, 1:2]) / d1
           + gd[:, 0:1]) / temp
    zout = ((SQ[:, 2:3] + Qb[:, 2:3] - SQ[:, 3:4] - Qb[:, 3:4]) / d1
            + gd[:, 1:2]) / temp
    pin = 1.0 / (1.0 + jnp.exp(-zin))
    pout = 1.0 / (1.0 + jnp.exp(-zout))
    y = pout * v_r[...]
    y_r[...] = jnp.stack([y[:, :128], y[:, 128:]], axis=0).astype(jnp.bfloat16)
    p_r[...] = jnp.concatenate(
        [pout, pin, jnp.zeros((BN, 6), jnp.float32)], axis=1)


def _tc6_body(ya_r, ea_r, p_r, v_r, wedge_r, wenv_r, be_r, lg_r, lb_r, o_r):
    ya = ya_r[...].astype(jnp.float32)
    Sv = jnp.concatenate([ya[0], ya[1]], axis=1)
    ea = ea_r[...]
    ea2 = ea[0] + ea[1]
    Sea = ea2[:, 0:16]
    Sp = ea2[:, 16:17]
    pin = p_r[...][:, 1:2]
    wee = _dot(wedge_r[...], wenv_r[...])
    num = pin * (Sv + _dot(Sea, wee)) + v_r[...]
    h = num / (pin * Sp + 1.0) + be_r[...]
    mu = jnp.mean(h, axis=1, keepdims=True)
    var = jnp.mean((h - mu) ** 2, axis=1, keepdims=True)
    o_r[...] = (h - mu) / jnp.sqrt(var + 1e-5) * lg_r[...] + lb_r[...]


def _full(shape):
    return pl.BlockSpec(shape, lambda i: tuple(0 for _ in shape))


def _rows(shape):
    if len(shape) == 2:
        return pl.BlockSpec(shape, lambda i: (i, 0))
    return pl.BlockSpec(shape, lambda i: (0, i, 0))


_tc_pass2 = pl.pallas_call(
    _tc2_body,
    grid=(GRID,),
    in_specs=[_rows((BN, 256)), _rows((2, BN, 128)), _rows((1, BN, 8)),
              _full((256, 256)), _full((256, 2)), _full((256, 256)),
              _full((256, 2)), _full((256, 1)), _full((256, 256))],
    out_specs=[_rows((BN, 16)), _rows((BN, 256))],
    out_shape=[jax.ShapeDtypeStruct((N, 16), jnp.float32),
               jax.ShapeDtypeStruct((N, 256), jnp.float32)],
)

_tc_pass4 = pl.pallas_call(
    _tc4_body,
    grid=(GRID,),
    in_specs=[_rows((BN, 16)), _rows((2, BN, 16)), _rows((1, BN, 8)),
              _rows((BN, 8)), _rows((BN, 256))],
    out_specs=[_rows((2, BN, 128)), _rows((BN, 8))],
    out_shape=[jax.ShapeDtypeStruct((2, N, 128), jnp.bfloat16),
               jax.ShapeDtypeStruct((N, 8), jnp.float32)],
)

_tc_pass6 = pl.pallas_call(
    _tc6_body,
    grid=(GRID,),
    in_specs=[_rows((2, BN, 128)), _rows((2, BN, 32)), _rows((BN, 8)),
              _rows((BN, 256)), _full((16, 256)), _full((256, 256)),
              _full((1, 256)), _full((1, 256)), _full((1, 256))],
    out_specs=[_rows((BN, 256))],
    out_shape=[jax.ShapeDtypeStruct((N, 256), jnp.float32)],
)


def kernel(x, edge_index, edge_attr, W_env, b_env, W_edge, W_in1, W_in2,
           W_out1, W_out2, W_temp, ln_gamma, ln_beta):
    f32 = jnp.float32
    src = edge_index[0]
    dst = edge_index[1]
    pad = EP - E
    srcp = jnp.concatenate([src, jnp.zeros((pad,), jnp.int32)])
    dstp = jnp.concatenate([dst, jnp.full((pad,), N, jnp.int32)])
    sidx = srcp.reshape(EROWS, K)
    didx = dstp.reshape(EROWS, K)
    sidx1 = srcp.reshape(EROWS1, K1)
    didx1 = dstp.reshape(EROWS1, K1)

    zrows = jnp.zeros((ROWS_PT, 128), f32)
    zcnt = jnp.zeros((ROWS_PT, 8), f32)
    zq = jnp.zeros((ROWS_PT, 16), f32)
    zea = jnp.zeros((ROWS_PT, 32), f32)
    onesr = jnp.concatenate(
        [jnp.ones((K1, 1), f32), jnp.zeros((K1, 7), f32)], axis=1)

    agg_s, cnt_s = _sc_pass1(x[:, :128], x[:, 128:], sidx1, didx1,
                             zrows, zcnt, onesr)

    Q, v = _tc_pass2(x, agg_s, cnt_s, W_in1, W_in2, W_out1,
                     W_out2, W_temp, W_env)

    (sq_s,) = _sc_pass3(Q, sidx, didx, zq)

    gkey = jax.random.key(42)
    k1, k2 = jax.random.split(gkey)
    u_in = jax.random.uniform(k1, (N, 2), f32, 1e-6, 1.0 - 1e-6)
    u_out = jax.random.uniform(k2, (N, 2), f32, 1e-6, 1.0 - 1e-6)
    g_in = -jnp.log(-jnp.log(u_in))
    g_out = -jnp.log(-jnp.log(u_out))
    gd = jnp.concatenate(
        [(g_in[:, 0] - g_in[:, 1])[:, None],
         (g_out[:, 0] - g_out[:, 1])[:, None],
         jnp.zeros((N, 6), f32)], axis=1)
    ys, P = _tc_pass4(Q, sq_s, cnt_s, gd, v)

    ptab = P[:, 0]
    eat = jnp.pad(edge_attr, ((0, pad), (0, 0))).T
    (ya_s,) = _sc_pass5a(ys[0], ys[1], sidx1, didx1,
                         jnp.zeros((ROWS_PT, 128), jnp.bfloat16))
    (eaa_s,) = _sc_pass5b(sidx, didx, ptab, eat, zea)

    out = _tc_pass6(ya_s, eaa_s, P, v, W_edge, W_env,
                    b_env.reshape(1, 256), ln_gamma.reshape(1, 256),
                    ln_beta.reshape(1, 256))
    return out[0]


# speedup vs baseline: 1.0953x; 1.0224x over previous
import functools

import jax
import jax.numpy as jnp
from jax import lax
from jax.experimental import pallas as pl
from jax.experimental.pallas import tpu as pltpu
from jax.experimental.pallas import tpu_sc as plsc

N = 10000
E = 160000
TAU0 = 0.1

NS = 16
NP = 10112
ROWS_PT = NP // NS
EP = 163840
K = 128
K1 = 40
EPT1 = EP // NS
NCH1 = EPT1 // K1
EPT2 = EP // (2 * NS)
NCH2 = EPT2 // K
EROWS = EP // K
EROWS1 = EP // K1
NCH5 = EPT2 // K1


def _ring4(nch, issue_gather, wait_gather, scatter_start, scatter_wait):
    for i in range(min(4, nch)):
        issue_gather(i, i)

    def quad(i4, carry):
        for q in range(4):
            i = i4 * 4 + q
            nb = (q + 1) % 4

            @pl.when(i >= 3)
            def _prep(i=i, nb=nb):
                scatter_wait(i - 3, nb)

                @pl.when(i + 1 < nch)
                def _ig(i=i, nb=nb):
                    issue_gather(i + 1, nb)

            wait_gather(i, q)
            scatter_start(i, q)
        return carry

    lax.fori_loop(0, nch // 4, quad, 0)
    for k in range(3):
        i = nch - 3 + k
        scatter_wait(i, i % 4)

_mesh = plsc.VectorSubcoreMesh(core_axis_name="c", subcore_axis_name="s")
_sc_params = pltpu.CompilerParams(use_tc_tiling_on_sc=False,
                                  needs_layout_passes=False)


def _p1_body(t1a, t1b, sidx, didx, zrows, zcnt, onesr, agg_o, cnt_o,
             src_all, dst_all, b0, b1, b2, b3, ones_v, acc, accc,
             g0, g1, g2, g3, s0, s1, s2, s3):
    c = lax.axis_index("c")
    s = lax.axis_index("s")
    r0 = s * ROWS_PT
    bufs = [b0, b1, b2, b3]
    gsem = [g0, g1, g2, g3]
    ssem = [s0, s1, s2, s3]
    pltpu.sync_copy(zrows, acc.at[pl.ds(r0, ROWS_PT)])
    pltpu.sync_copy(zcnt, accc.at[pl.ds(r0, ROWS_PT)])
    pltpu.sync_copy(onesr, ones_v)
    pltpu.sync_copy(sidx.at[pl.ds(s * NCH1, NCH1)], src_all)
    pltpu.sync_copy(didx.at[pl.ds(s * NCH1, NCH1)], dst_all)
    plsc.subcore_barrier()

    def issue_gather(i, q):
        @pl.when(c == 0)
        def _a():
            pltpu.async_copy(t1a.at[src_all.at[i]], bufs[q], gsem[q])

        @pl.when(c == 1)
        def _b():
            pltpu.async_copy(t1b.at[src_all.at[i]], bufs[q], gsem[q])

    def wait_gather(i, q):
        pltpu.make_async_copy(t1a.at[src_all.at[i]], bufs[q], gsem[q]).wait()

    def scatter_start(i, q):
        pltpu.async_copy(bufs[q], acc.at[dst_all.at[i]], ssem[q], add=True)
        pltpu.async_copy(ones_v, accc.at[dst_all.at[i]], ssem[q], add=True)

    def scatter_wait(i, q):
        pltpu.make_async_copy(bufs[q], acc.at[dst_all.at[i]], ssem[q]).wait()
        pltpu.make_async_copy(ones_v, accc.at[dst_all.at[i]], ssem[q]).wait()

    _ring4(NCH1, issue_gather, wait_gather, scatter_start, scatter_wait)
    plsc.subcore_barrier()
    pltpu.sync_copy(acc.at[pl.ds(r0, ROWS_PT)], agg_o.at[c, pl.ds(r0, ROWS_PT)])
    pltpu.sync_copy(accc.at[pl.ds(r0, ROWS_PT)], cnt_o.at[c, pl.ds(r0, ROWS_PT)])


_sc_pass1 = pl.kernel(
    _p1_body, mesh=_mesh, compiler_params=_sc_params,
    out_type=[jax.ShapeDtypeStruct((2, NP, 128), jnp.float32),
              jax.ShapeDtypeStruct((2, NP, 8), jnp.float32)],
    scratch_types=(
        [pltpu.VMEM((NCH1, K1), jnp.int32),
         pltpu.VMEM((NCH1, K1), jnp.int32)]
        + [pltpu.VMEM((K1, 128), jnp.float32)] * 4
        + [pltpu.VMEM((K1, 8), jnp.float32),
           pltpu.VMEM_SHARED((NP, 128), jnp.float32),
           pltpu.VMEM_SHARED((NP, 8), jnp.float32)]
        + [pltpu.SemaphoreType.DMA] * 8
    ))


def _p3_body(qtab, sidx, didx, zq, sq_o, src_all, dst_all, b0, b1, b2, b3,
             acc, g0, g1, g2, g3, s0, s1, s2, s3):
    c = lax.axis_index("c")
    s = lax.axis_index("s")
    r0 = s * ROWS_PT
    w = c * NS + s
    bufs = [b0, b1, b2, b3]
    gsem = [g0, g1, g2, g3]
    ssem = [s0, s1, s2, s3]
    pltpu.sync_copy(zq, acc.at[pl.ds(r0, ROWS_PT)])
    pltpu.sync_copy(sidx.at[pl.ds(w * NCH2, NCH2)], src_all)
    pltpu.sync_copy(didx.at[pl.ds(w * NCH2, NCH2)], dst_all)
    plsc.subcore_barrier()

    def issue_gather(i, q):
        pltpu.async_copy(qtab.at[src_all.at[i]], bufs[q], gsem[q])

    def wait_gather(i, q):
        pltpu.make_async_copy(qtab.at[src_all.at[i]], bufs[q], gsem[q]).wait()

    def scatter_start(i, q):
        pltpu.async_copy(bufs[q], acc.at[dst_all.at[i]], ssem[q], add=True)

    def scatter_wait(i, q):
        pltpu.make_async_copy(bufs[q], acc.at[dst_all.at[i]], ssem[q]).wait()

    _ring4(NCH2, issue_gather, wait_gather, scatter_start, scatter_wait)
    plsc.subcore_barrier()
    pltpu.sync_copy(acc.at[pl.ds(r0, ROWS_PT)], sq_o.at[c, pl.ds(r0, ROWS_PT)])


_sc_pass3 = pl.kernel(
    _p3_body, mesh=_mesh, compiler_params=_sc_params,
    out_type=[jax.ShapeDtypeStruct((2, NP, 16), jnp.float32)],
    scratch_types=(
        [pltpu.VMEM((NCH2, K), jnp.int32),
         pltpu.VMEM((NCH2, K), jnp.int32)]
        + [pltpu.VMEM((K, 16), jnp.float32)] * 4
        + [pltpu.VMEM_SHARED((NP, 16), jnp.float32)]
        + [pltpu.SemaphoreType.DMA] * 8
    ))


def _p5a_body(yta, ytb, sidx, didx, zrows, ya_o,
              src_all, dst_all, b0, b1, b2, b3, acc,
              g0, g1, g2, g3, s0, s1, s2, s3):
    c = lax.axis_index("c")
    s = lax.axis_index("s")
    r0 = s * ROWS_PT
    bufs = [b0, b1, b2, b3]
    gsem = [g0, g1, g2, g3]
    ssem = [s0, s1, s2, s3]
    pltpu.sync_copy(zrows, acc.at[pl.ds(r0, ROWS_PT)])
    pltpu.sync_copy(sidx.at[pl.ds(s * NCH1, NCH1)], src_all)
    pltpu.sync_copy(didx.at[pl.ds(s * NCH1, NCH1)], dst_all)
    plsc.subcore_barrier()

    def issue_gather(i, q):
        @pl.when(c == 0)
        def _a():
            pltpu.async_copy(yta.at[src_all.at[i]], bufs[q], gsem[q])

        @pl.when(c == 1)
        def _b():
            pltpu.async_copy(ytb.at[src_all.at[i]], bufs[q], gsem[q])

    def wait_gather(i, q):
        pltpu.make_async_copy(yta.at[src_all.at[i]], bufs[q], gsem[q]).wait()

    def scatter_start(i, q):
        pltpu.async_copy(bufs[q], acc.at[dst_all.at[i]], ssem[q], add=True)

    def scatter_wait(i, q):
        pltpu.make_async_copy(bufs[q], acc.at[dst_all.at[i]], ssem[q]).wait()

    _ring4(NCH1, issue_gather, wait_gather, scatter_start, scatter_wait)
    plsc.subcore_barrier()
    pltpu.sync_copy(acc.at[pl.ds(r0, ROWS_PT)], ya_o.at[c, pl.ds(r0, ROWS_PT)])


_sc_pass5a = pl.kernel(
    _p5a_body, mesh=_mesh, compiler_params=_sc_params,
    out_type=[jax.ShapeDtypeStruct((2, NP, 128), jnp.bfloat16)],
    scratch_types=(
        [pltpu.VMEM((NCH1, K1), jnp.int32),
         pltpu.VMEM((NCH1, K1), jnp.int32)]
        + [pltpu.VMEM((K1, 128), jnp.bfloat16)] * 4
        + [pltpu.VMEM_SHARED((NP, 128), jnp.bfloat16)]
        + [pltpu.SemaphoreType.DMA] * 8
    ))


def _p5b_body(sidx, didx, ptab, eat, zea, eaa_o,
              esrc_all, edst_all, ea0, ea1, st0, st1, ptab_v, accea,
              e0s, e1s, s0s, s1s):
    c = lax.axis_index("c")
    s = lax.axis_index("s")
    r0 = s * ROWS_PT
    w = c * NS + s
    eat_v = [ea0, ea1]
    stage = [st0, st1]
    esem = [e0s, e1s]
    ssem = [s0s, s1s]
    pltpu.sync_copy(zea, accea.at[pl.ds(r0, ROWS_PT)])
    pltpu.sync_copy(zea.at[pl.ds(0, K)], st0)
    pltpu.sync_copy(zea.at[pl.ds(0, K)], st1)
    pltpu.sync_copy(ptab, ptab_v)
    pltpu.sync_copy(sidx.at[pl.ds(w * NCH2, NCH2)], esrc_all)
    pltpu.sync_copy(didx.at[pl.ds(w * NCH2, NCH2)], edst_all)
    plsc.subcore_barrier()

    iot = lax.iota(jnp.int32, 16)
    base_e = w * EPT2

    def eload(g, q):
        pltpu.async_copy(eat.at[:, pl.ds(base_e + g * K, K)], eat_v[q],
                         esem[q])

    def ewait(g, q):
        pltpu.make_async_copy(eat.at[:, pl.ds(base_e + g * K, K)], eat_v[q],
                              esem[q]).wait()

    def swait(i, q):
        pltpu.make_async_copy(stage[q], accea.at[edst_all.at[i]],
                              ssem[q]).wait()

    def compute(g, q):
        def group(g2, carry2):
            b = g2 * 16
            idxv = esrc_all[g, pl.ds(b, 16)]
            pv = plsc.load_gather(ptab_v, [idxv])
            rows16 = b + iot
            for j in range(16):
                colv = eat_v[q][j, pl.ds(b, 16)]
                plsc.store_scatter(
                    stage[q], [rows16, jnp.full((16,), j, jnp.int32)],
                    colv * pv)
            plsc.store_scatter(
                stage[q], [rows16, jnp.full((16,), 16, jnp.int32)], pv)
            return carry2

        lax.fori_loop(0, K // 16, group, 0)
        pltpu.async_copy(stage[q], accea.at[edst_all.at[g]], ssem[q],
                         add=True)

    eload(0, 0)

    def pair(p, carry):
        g0 = 2 * p
        eload(g0 + 1, 1)
        ewait(g0, 0)

        @pl.when(g0 >= 2)
        def _w0():
            swait(g0 - 2, 0)

        compute(g0, 0)

        @pl.when(g0 + 2 < NCH2)
        def _l2():
            eload(g0 + 2, 0)

        ewait(g0 + 1, 1)

        @pl.when(g0 >= 1)
        def _w1():
            swait(g0 - 1, 1)

        compute(g0 + 1, 1)
        return carry

    lax.fori_loop(0, NCH2 // 2, pair, 0)
    swait(NCH2 - 2, 0)
    swait(NCH2 - 1, 1)
    plsc.subcore_barrier()
    pltpu.sync_copy(accea.at[pl.ds(r0, ROWS_PT)], eaa_o.at[c, pl.ds(r0, ROWS_PT)])


_sc_pass5b = pl.kernel(
    _p5b_body, mesh=_mesh, compiler_params=_sc_params,
    out_type=[jax.ShapeDtypeStruct((2, NP, 32), jnp.float32)],
    scratch_types=(
        [pltpu.VMEM((NCH2, K), jnp.int32),
         pltpu.VMEM((NCH2, K), jnp.int32)]
        + [pltpu.VMEM((16, K), jnp.float32)] * 2
        + [pltpu.VMEM((K, 32), jnp.float32)] * 2
        + [pltpu.VMEM((N,), jnp.float32),
           pltpu.VMEM_SHARED((NP, 32), jnp.float32)]
        + [pltpu.SemaphoreType.DMA] * 4
    ))


BN = 1000
GRID = N // BN


def _dot(a, b):
    return lax.dot_general(a, b, (((1,), (0,)), ((), ())),
                           precision=lax.Precision.HIGHEST,
                           preferred_element_type=jnp.float32)


def _dot_fast(a, b):
    return lax.dot_general(a, b, (((1,), (0,)), ((), ())),
                           preferred_element_type=jnp.float32)


def _tc2_body(x_r, agg_r, cnt_r, w1_r, w2_r, w3_r, w4_r, wt_r, we_r,
              q_r, v_r):
    x = x_r[...]
    agg = agg_r[...]
    A = (jnp.concatenate([agg[0], agg[1]], axis=1) + x) \
        / (cnt_r[...][0][:, 0:1] + 1.0)
    hin = jnp.maximum(_dot(A, w1_r[...]), 0.0)
    hout = jnp.maximum(_dot(A, w3_r[...]), 0.0)
    qin = _dot(hin, w2_r[...])
    qout = _dot(hout, w4_r[...])
    tz = _dot(A, wt_r[...])
    temp = jnp.maximum(tz, 0.0) + jnp.log(1.0 + jnp.exp(-jnp.abs(tz))) + TAU0
    q_r[...] = jnp.concatenate(
        [qin, qout, temp, jnp.zeros((BN, 11), jnp.float32)], axis=1)
    v_r[...] = _dot_fast(x, we_r[...])


def _tc4_body(q_r, sq_r, cnt_r, gd_r, v_r, y_r, p_r):
    Qb = q_r[...]
    sq = sq_r[...]
    SQ = sq[0] + sq[1]
    d1 = cnt_r[...][0][:, 0:1] + 1.0
    gd = gd_r[...]
    temp = Qb[:, 4:5]
    zin = ((SQ[:, 0:1] + Qb[:, 0:1] - SQ[:, 1:2] - Qb[:, 1:2]) / d1
           + gd[:, 0:1]) / temp
    zout = ((SQ[:, 2:3] + Qb[:, 2:3] - SQ[:, 3:4] - Qb[:, 3:4]) / d1
            + gd[:, 1:2]) / temp
    pin = 1.0 / (1.0 + jnp.exp(-zin))
    pout = 1.0 / (1.0 + jnp.exp(-zout))
    y = pout * v_r[...]
    y_r[...] = jnp.stack([y[:, :128], y[:, 128:]], axis=0).astype(jnp.bfloat16)
    p_r[...] = jnp.concatenate(
        [pout, pin, jnp.zeros((BN, 6), jnp.float32)], axis=1)


def _tc6_body(ya_r, ea_r, p_r, v_r, wedge_r, wenv_r, be_r, lg_r, lb_r, o_r):
    ya = ya_r[...].astype(jnp.float32)
    Sv = jnp.concatenate([ya[0], ya[1]], axis=1)
    ea = ea_r[...]
    ea2 = ea[0] + ea[1]
    Sea = ea2[:, 0:16]
    Sp = ea2[:, 16:17]
    pin = p_r[...][:, 1:2]
    wee = _dot_fast(wedge_r[...], wenv_r[...])
    num = pin * (Sv + _dot_fast(Sea, wee)) + v_r[...]
    h = num / (pin * Sp + 1.0) + be_r[...]
    mu = jnp.mean(h, axis=1, keepdims=True)
    var = jnp.mean((h - mu) ** 2, axis=1, keepdims=True)
    o_r[...] = (h - mu) / jnp.sqrt(var + 1e-5) * lg_r[...] + lb_r[...]


def _full(shape):
    return pl.BlockSpec(shape, lambda i: tuple(0 for _ in shape))


def _rows(shape):
    if len(shape) == 2:
        return pl.BlockSpec(shape, lambda i: (i, 0))
    return pl.BlockSpec(shape, lambda i: (0, i, 0))


_tc_pass2 = pl.pallas_call(
    _tc2_body,
    grid=(GRID,),
    in_specs=[_rows((BN, 256)), _rows((2, BN, 128)), _rows((1, BN, 8)),
              _full((256, 256)), _full((256, 2)), _full((256, 256)),
              _full((256, 2)), _full((256, 1)), _full((256, 256))],
    out_specs=[_rows((BN, 16)), _rows((BN, 256))],
    out_shape=[jax.ShapeDtypeStruct((N, 16), jnp.float32),
               jax.ShapeDtypeStruct((N, 256), jnp.float32)],
)

_tc_pass4 = pl.pallas_call(
    _tc4_body,
    grid=(GRID,),
    in_specs=[_rows((BN, 16)), _rows((2, BN, 16)), _rows((1, BN, 8)),
              _rows((BN, 8)), _rows((BN, 256))],
    out_specs=[_rows((2, BN, 128)), _rows((BN, 8))],
    out_shape=[jax.ShapeDtypeStruct((2, N, 128), jnp.bfloat16),
               jax.ShapeDtypeStruct((N, 8), jnp.float32)],
)

_tc_pass6 = pl.pallas_call(
    _tc6_body,
    grid=(GRID,),
    in_specs=[_rows((2, BN, 128)), _rows((2, BN, 32)), _rows((BN, 8)),
              _rows((BN, 256)), _full((16, 256)), _full((256, 256)),
              _full((1, 256)), _full((1, 256)), _full((1, 256))],
    out_specs=[_rows((BN, 256))],
    out_shape=[jax.ShapeDtypeStruct((N, 256), jnp.float32)],
)


def kernel(x, edge_index, edge_attr, W_env, b_env, W_edge, W_in1, W_in2,
           W_out1, W_out2, W_temp, ln_gamma, ln_beta):
    f32 = jnp.float32
    src = edge_index[0]
    dst = edge_index[1]
    pad = EP - E
    srcp = jnp.concatenate([src, jnp.zeros((pad,), jnp.int32)])
    dstp = jnp.concatenate([dst, jnp.full((pad,), N, jnp.int32)])
    sidx = srcp.reshape(EROWS, K)
    didx = dstp.reshape(EROWS, K)
    sidx1 = srcp.reshape(EROWS1, K1)
    didx1 = dstp.reshape(EROWS1, K1)

    zrows = jnp.zeros((ROWS_PT, 128), f32)
    zcnt = jnp.zeros((ROWS_PT, 8), f32)
    zq = jnp.zeros((ROWS_PT, 16), f32)
    zea = jnp.zeros((ROWS_PT, 32), f32)
    onesr = jnp.concatenate(
        [jnp.ones((K1, 1), f32), jnp.zeros((K1, 7), f32)], axis=1)

    agg_s, cnt_s = _sc_pass1(x[:, :128], x[:, 128:], sidx1, didx1,
                             zrows, zcnt, onesr)

    Q, v = _tc_pass2(x, agg_s, cnt_s, W_in1, W_in2, W_out1,
                     W_out2, W_temp, W_env)

    (sq_s,) = _sc_pass3(Q, sidx, didx, zq)

    gkey = jax.random.key(42)
    k1, k2 = jax.random.split(gkey)
    u_in = jax.random.uniform(k1, (N, 2), f32, 1e-6, 1.0 - 1e-6)
    u_out = jax.random.uniform(k2, (N, 2), f32, 1e-6, 1.0 - 1e-6)
    g_in = -jnp.log(-jnp.log(u_in))
    g_out = -jnp.log(-jnp.log(u_out))
    gd = jnp.concatenate(
        [(g_in[:, 0] - g_in[:, 1])[:, None],
         (g_out[:, 0] - g_out[:, 1])[:, None],
         jnp.zeros((N, 6), f32)], axis=1)
    ys, P = _tc_pass4(Q, sq_s, cnt_s, gd, v)

    ptab = P[:, 0]
    eat = jnp.pad(edge_attr, ((0, pad), (0, 0))).T
    (ya_s,) = _sc_pass5a(ys[0], ys[1], sidx1, didx1,
                         jnp.zeros((ROWS_PT, 128), jnp.bfloat16))
    (eaa_s,) = _sc_pass5b(sidx, didx, ptab, eat, zea)

    out = _tc_pass6(ya_s, eaa_s, P, v, W_edge, W_env,
                    b_env.reshape(1, 256), ln_gamma.reshape(1, 256),
                    ln_beta.reshape(1, 256))
    return out[0]
